# baseline (device time: 186746 ns/iter reference)
import jax
import jax.numpy as jnp
from jax import lax
from jax.experimental import pallas as pl
from jax.experimental.pallas import tpu as pltpu

N_DEV = 4
N_RS = N_DEV - 1
N_RINGS = 4


def kernel(x, w_mat, scale_x, scale_w):
    m, k_per = x.shape
    _, n = w_mat.shape
    m_per = m // N_DEV
    n_q = n // N_RINGS

    def body(x_hbm, w_ref, sx_ref, sw_ref, out_ref,
             xchunk, stage, rs_buf, load_sem, send_sems, recv_sems,
             cred_sems):
        my = lax.axis_index("i")
        left = (my - 1) % N_DEV
        right = (my + 1) % N_DEV

        def rows(c):
            return pl.ds(c * m_per, m_per)

        def cols(q):
            return pl.ds(q * n_q, n_q)

        def up(q):
            return left if q % 2 == 0 else right

        def down(q):
            return right if q % 2 == 0 else left

        def own(q):
            return ((my + 1) if q % 2 == 0 else (my - 1)) % N_DEV

        def slot(h):
            return h % 2

        barrier_sem = pltpu.get_barrier_semaphore()
        for nbr in (left, right):
            pl.semaphore_signal(
                barrier_sem, inc=1,
                device_id=(nbr,), device_id_type=pl.DeviceIdType.MESH,
            )
        pl.semaphore_wait(barrier_sem, 2)

        def send(q, h, src):
            rdma = pltpu.make_async_remote_copy(
                src_ref=src,
                dst_ref=rs_buf.at[q, slot(h)],
                send_sem=send_sems.at[q, h],
                recv_sem=recv_sems.at[q, h],
                device_id=(down(q),),
                device_id_type=pl.DeviceIdType.MESH,
            )
            rdma.start()
            return rdma

        def signal_cred(q, p):
            pl.semaphore_signal(
                cred_sems.at[q, p], inc=1,
                device_id=(up(q),), device_id_type=pl.DeviceIdType.MESH,
            )

        def wait_cred(q, p):
            pl.semaphore_wait(cred_sems.at[q, p], 1)

        def load(c):
            cp = pltpu.make_async_copy(
                x_hbm.at[rows(c), :], xchunk, load_sem)
            cp.start()
            return cp

        load(my).wait()
        for q in range(N_RINGS):
            r = jnp.dot(xchunk[...], w_ref[:, q * n_q:(q + 1) * n_q],
                        preferred_element_type=jnp.float32)
            out_ref[rows(my), cols(q)] = r
            stage[q] = r.astype(jnp.bfloat16)
        pend = [send(q, 0, stage.at[q]) for q in range(N_RINGS)]
        for c_off in (-1, 1, 2):
            c = (my + c_off) % N_DEV
            load(c).wait()
            out_ref[rows(c), :] = jnp.dot(
                xchunk[...], w_ref[...],
                preferred_element_type=jnp.float32)

        for s in range(N_RS):
            for q in range(N_RINGS):
                pend[q].wait()
                recv_c = ((my - 1 - s) if q % 2 == 0 else
                          (my + 1 + s)) % N_DEV
                v = (out_ref[rows(recv_c), cols(q)] +
                     rs_buf[q, slot(s)].astype(jnp.float32))
                out_ref[rows(recv_c), cols(q)] = v
                signal_cred(q, slot(s))
                if s + 1 < N_RS:
                    stage[q] = v.astype(jnp.bfloat16)
                    if slot(s + 1) == slot(0):
                        wait_cred(q, slot(s + 1))
                    pend[q] = send(q, s + 1, stage.at[q])

        scale = sx_ref[0] * sw_ref[0]
        for q in range(N_RINGS):
            v = jnp.maximum(out_ref[rows(own(q)), cols(q)] * scale, 0.0)
            out_ref[rows(own(q)), cols(q)] = v
            stage[q] = v.astype(jnp.bfloat16)
            wait_cred(q, slot(3))
            pend[q] = send(q, 3, stage.at[q])

        for t in range(N_RS):
            h = N_RS + t
            for q in range(N_RINGS):
                pend[q].wait()
                recv_c = ((my - t) if q % 2 == 0 else (my + t)) % N_DEV
                out_ref[rows(recv_c), cols(q)] = (
                    rs_buf[q, slot(h)].astype(jnp.float32))
                if t == 1:
                    signal_cred(q, slot(3))
                if t + 1 < N_RS:
                    wait_cred(q, slot(h + 1))
                    pend[q] = send(q, h + 1, rs_buf.at[q, slot(h)])

    return pl.pallas_call(
        body,
        out_shape=jax.ShapeDtypeStruct((m, n), jnp.float32),
        in_specs=[
            pl.BlockSpec(memory_space=pl.ANY),
            pl.BlockSpec(memory_space=pltpu.VMEM),
            pl.BlockSpec(memory_space=pltpu.SMEM),
            pl.BlockSpec(memory_space=pltpu.SMEM),
        ],
        out_specs=pl.BlockSpec(memory_space=pltpu.VMEM),
        scratch_shapes=[
            pltpu.VMEM((m_per, k_per), jnp.float32),
            pltpu.VMEM((N_RINGS, m_per, n_q), jnp.bfloat16),
            pltpu.VMEM((N_RINGS, 2, m_per, n_q), jnp.bfloat16),
            pltpu.SemaphoreType.DMA,
            pltpu.SemaphoreType.DMA((N_RINGS, 2 * N_RS)),
            pltpu.SemaphoreType.DMA((N_RINGS, 2 * N_RS)),
            pltpu.SemaphoreType.REGULAR((N_RINGS, 2)),
        ],
        compiler_params=pltpu.CompilerParams(
            collective_id=0, vmem_limit_bytes=64 * 1024 * 1024),
    )(x, w_mat, scale_x, scale_w)


# device time: 176041 ns/iter; 1.0608x vs baseline; 1.0608x over previous
import jax
import jax.numpy as jnp
from jax import lax
from jax.experimental import pallas as pl
from jax.experimental.pallas import tpu as pltpu

N_DEV = 4
N_RS = N_DEV - 1
N_RINGS = 4


def kernel(x, w_mat, scale_x, scale_w):
    m, k_per = x.shape
    _, n = w_mat.shape
    m_per = m // N_DEV
    n_q = n // N_RINGS

    def body(x_hbm, w_ref, sx_ref, sw_ref, out_hbm,
             xchunk, w_bf, qbuf, rs_buf, stage, load_sems, store_sems,
             send_sems, recv_sems):
        my = lax.axis_index("i")
        left = (my - 1) % N_DEV
        right = (my + 1) % N_DEV

        def rows(c):
            return pl.ds(c * m_per, m_per)

        def cols(q):
            return pl.ds(q * n_q, n_q)

        barrier_sem = pltpu.get_barrier_semaphore()
        for nbr in (left, right):
            pl.semaphore_signal(
                barrier_sem, inc=1,
                device_id=(nbr,), device_id_type=pl.DeviceIdType.MESH,
            )
        pl.semaphore_wait(barrier_sem, 2)

        def load(c, slot):
            cp = pltpu.make_async_copy(
                x_hbm.at[rows(c), :], xchunk.at[slot], load_sems.at[slot])
            cp.start()
            return cp

        def compute(c, slot):
            xc = xchunk[slot].astype(jnp.bfloat16)
            for q in range(N_RINGS):
                qbuf[q, rows(c), :] = jnp.dot(
                    xc, w_bf[:, q * n_q:(q + 1) * n_q],
                    preferred_element_type=jnp.float32).astype(jnp.bfloat16)

        def rs_rdma(q, s):
            rightward = q % 2 == 0
            nbr = right if rightward else left
            send_c = (my - s) % N_DEV if rightward else (my + s) % N_DEV
            rdma = pltpu.make_async_remote_copy(
                src_ref=qbuf.at[q, rows(send_c), :],
                dst_ref=rs_buf.at[q, s],
                send_sem=send_sems.at[q, s],
                recv_sem=recv_sems.at[q, s],
                device_id=(nbr,),
                device_id_type=pl.DeviceIdType.MESH,
            )
            rdma.start()
            return rdma

        pending = [None] * N_RINGS
        cp = load(my, 0)
        w_bf[...] = w_ref[...].astype(jnp.bfloat16)
        cp.wait()
        cp = load((my - 1) % N_DEV, 1)
        xc0 = xchunk[0].astype(jnp.bfloat16)
        for q in range(N_RINGS):
            qbuf[q, rows(my), :] = jnp.dot(
                xc0, w_bf[:, q * n_q:(q + 1) * n_q],
                preferred_element_type=jnp.float32).astype(jnp.bfloat16)
            pending[q] = rs_rdma(q, 0)
        cp.wait()
        cp = load((my + 1) % N_DEV, 0)
        compute((my - 1) % N_DEV, 1)
        cp.wait()
        cp = load((my + 2) % N_DEV, 1)
        compute((my + 1) % N_DEV, 0)
        cp.wait()
        compute((my + 2) % N_DEV, 1)

        for s in range(N_RS):
            for q in range(N_RINGS):
                pending[q].wait()
                recv_c = ((my - 1 - s) if q % 2 == 0 else
                          (my + 1 + s)) % N_DEV
                qbuf[q, rows(recv_c), :] += rs_buf[q, s]
                if s + 1 < N_RS:
                    pending[q] = rs_rdma(q, s + 1)

        scale = (sx_ref[0] * sw_ref[0]).astype(jnp.bfloat16)

        def own(q):
            return ((my + 1) if q % 2 == 0 else (my - 1)) % N_DEV

        for q in range(N_RINGS):
            v = qbuf[q, rows(own(q)), :]
            qbuf[q, rows(own(q)), :] = jnp.maximum(v * scale, 0)

        store_cps = []

        def store_quarter(c, q):
            k = len(store_cps)
            slot = k % 2
            if k >= 2:
                store_cps[k - 2].wait()
            stage[slot] = qbuf[q, rows(c), :].astype(jnp.float32)
            cp = pltpu.make_async_copy(
                stage.at[slot], out_hbm.at[rows(c), cols(q)],
                store_sems.at[slot])
            cp.start()
            store_cps.append(cp)

        def ag_rdma(q, t):
            rightward = q % 2 == 0
            nbr = right if rightward else left
            send_c = ((my + 1 - t) if rightward else (my - 1 + t)) % N_DEV
            rdma = pltpu.make_async_remote_copy(
                src_ref=qbuf.at[q, rows(send_c), :],
                dst_ref=qbuf.at[q, rows(send_c), :],
                send_sem=send_sems.at[q, N_RS + t],
                recv_sem=recv_sems.at[q, N_RS + t],
                device_id=(nbr,),
                device_id_type=pl.DeviceIdType.MESH,
            )
            rdma.start()
            return rdma

        pending = [ag_rdma(q, 0) for q in range(N_RINGS)]
        for q in range(N_RINGS):
            store_quarter(own(q), q)
        for t in range(N_RS):
            nxt = [None] * N_RINGS
            for q, rdma in enumerate(pending):
                rdma.wait()
                if t + 1 < N_RS:
                    nxt[q] = ag_rdma(q, t + 1)
                recv_c = ((my - t) if q % 2 == 0 else (my + t)) % N_DEV
                store_quarter(recv_c, q)
            pending = nxt

        store_cps[-2].wait()
        store_cps[-1].wait()

    return pl.pallas_call(
        body,
        out_shape=jax.ShapeDtypeStruct((m, n), jnp.float32),
        in_specs=[
            pl.BlockSpec(memory_space=pl.ANY),
            pl.BlockSpec(memory_space=pltpu.VMEM),
            pl.BlockSpec(memory_space=pltpu.SMEM),
            pl.BlockSpec(memory_space=pltpu.SMEM),
        ],
        out_specs=pl.BlockSpec(memory_space=pl.ANY),
        scratch_shapes=[
            pltpu.VMEM((2, m_per, k_per), jnp.float32),
            pltpu.VMEM((k_per, n), jnp.bfloat16),
            pltpu.VMEM((N_RINGS, m, n_q), jnp.bfloat16),
            pltpu.VMEM((N_RINGS, N_RS, m_per, n_q), jnp.bfloat16),
            pltpu.VMEM((2, m_per, n_q), jnp.float32),
            pltpu.SemaphoreType.DMA((2,)),
            pltpu.SemaphoreType.DMA((2,)),
            pltpu.SemaphoreType.DMA((N_RINGS, 2 * N_RS)),
            pltpu.SemaphoreType.DMA((N_RINGS, 2 * N_RS)),
        ],
        compiler_params=pltpu.CompilerParams(
            collective_id=0, vmem_limit_bytes=64 * 1024 * 1024),
    )(x, w_mat, scale_x, scale_w)
